# baseline (speedup 1.0000x reference)
import functools

import jax
import jax.numpy as jnp
from jax import lax
from jax.experimental import pallas as pl
from jax.experimental.pallas import tpu as pltpu
from jax.experimental.pallas import tpu_sc as plsc

N_NODES = 10000
N_EDGES = 320000
D_FEAT = 128

NC = 2
NS = 16
L = 16
NW = NC * NS
EPW = N_EDGES // NW
CHUNK = 80
NGROUP = CHUNK // L
NCHUNK = EPW // CHUNK


def _normalize_body(hf_ref, hs_ref, of_ref, os_ref):
    hf = hf_ref[...]
    hs = hs_ref[...]
    of_ref[...] = hf * lax.rsqrt(jnp.sum(hf * hf, axis=1, keepdims=True))
    os_ref[...] = hs * lax.rsqrt(jnp.sum(hs * hs, axis=1, keepdims=True))


def _normalize(h_first, h_second):
    rows = h_first.shape[0]
    blk = 2000
    grid = rows // blk
    spec = pl.BlockSpec((blk, D_FEAT), lambda i: (i, 0))
    return pl.pallas_call(
        _normalize_body,
        grid=(grid,),
        in_specs=[spec, spec],
        out_specs=[spec, spec],
        out_shape=[
            jax.ShapeDtypeStruct(h_first.shape, jnp.float32),
            jax.ShapeDtypeStruct(h_second.shape, jnp.float32),
        ],
    )(h_first, h_second)


def _sc_body(idx0_hbm, idx1_hbm, hf_hbm, hs_hbm, out_hbm,
             i0_all, i1_all, out_all,
             ra0, rb0, ra1, rb1, sa0, sb0, sa1, sb1):
    wid = lax.axis_index("s") * NC + lax.axis_index("c")
    base = wid * EPW
    pltpu.sync_copy(idx0_hbm.at[pl.ds(base, EPW)], i0_all)
    pltpu.sync_copy(idx1_hbm.at[pl.ds(base, EPW)], i1_all)

    lanes = lax.iota(jnp.int32, L)

    def issue(c, ra, rb, sa, sb):
        ia = i0_all.at[pl.ds(c * CHUNK, CHUNK)]
        ib = i1_all.at[pl.ds(c * CHUNK, CHUNK)]
        pltpu.async_copy(hf_hbm.at[ia], ra, sa)
        pltpu.async_copy(hs_hbm.at[ib], rb, sb)

    def drain(c, ra, rb, sa, sb):
        ia = i0_all.at[pl.ds(c * CHUNK, CHUNK)]
        ib = i1_all.at[pl.ds(c * CHUNK, CHUNK)]
        pltpu.make_async_copy(hf_hbm.at[ia], ra, sa).wait()
        pltpu.make_async_copy(hs_hbm.at[ib], rb, sb).wait()

    def compute(c, ra, rb):
        def group_body(g, carry2):
            rows = g * L + lanes
            a0 = jnp.zeros((L,), jnp.float32)
            a1 = jnp.zeros((L,), jnp.float32)
            a2 = jnp.zeros((L,), jnp.float32)
            a3 = jnp.zeros((L,), jnp.float32)
            for j in range(0, D_FEAT, 4):
                for k in range(4):
                    col = jnp.full((L,), j + k, jnp.int32)
                    p = plsc.load_gather(ra, [rows, col]) * plsc.load_gather(rb, [rows, col])
                    if k == 0:
                        a0 = a0 + p
                    elif k == 1:
                        a1 = a1 + p
                    elif k == 2:
                        a2 = a2 + p
                    else:
                        a3 = a3 + p
            out_all[pl.ds(c * CHUNK + g * L, L)] = (a0 + a1) + (a2 + a3)
            return carry2

        lax.fori_loop(0, NGROUP, group_body, 0)

    issue(0, ra0, rb0, sa0, sb0)
    issue(1, ra1, rb1, sa1, sb1)

    def pair_body(cc, carry):
        c0 = 2 * cc
        drain(c0, ra0, rb0, sa0, sb0)
        compute(c0, ra0, rb0)

        @pl.when(c0 + 2 < NCHUNK)
        def _():
            issue(c0 + 2, ra0, rb0, sa0, sb0)

        drain(c0 + 1, ra1, rb1, sa1, sb1)
        compute(c0 + 1, ra1, rb1)

        @pl.when(c0 + 3 < NCHUNK)
        def _():
            issue(c0 + 3, ra1, rb1, sa1, sb1)

        return carry

    lax.fori_loop(0, NCHUNK // 2, pair_body, 0)
    if NCHUNK % 2:
        c_last = NCHUNK - 1
        drain(c_last, ra0, rb0, sa0, sb0)
        compute(c_last, ra0, rb0)

    pltpu.sync_copy(out_all, out_hbm.at[pl.ds(base, EPW)])


@functools.partial(jax.jit, static_argnames=())
def _sc_edge_dots(idx0, idx1, hf_n, hs_n):
    mesh = plsc.VectorSubcoreMesh(core_axis_name="c", subcore_axis_name="s")
    return pl.kernel(
        _sc_body,
        out_type=jax.ShapeDtypeStruct((N_EDGES,), jnp.float32),
        mesh=mesh,
        compiler_params=pltpu.CompilerParams(needs_layout_passes=False),
        scratch_types=[
            pltpu.VMEM((EPW,), jnp.int32),
            pltpu.VMEM((EPW,), jnp.int32),
            pltpu.VMEM((EPW,), jnp.float32),
            pltpu.VMEM((CHUNK, D_FEAT), jnp.float32),
            pltpu.VMEM((CHUNK, D_FEAT), jnp.float32),
            pltpu.VMEM((CHUNK, D_FEAT), jnp.float32),
            pltpu.VMEM((CHUNK, D_FEAT), jnp.float32),
            pltpu.SemaphoreType.DMA,
            pltpu.SemaphoreType.DMA,
            pltpu.SemaphoreType.DMA,
            pltpu.SemaphoreType.DMA,
        ],
    )(idx0, idx1, hf_n, hs_n)


def kernel(edges_supervised, h_first, h_second):
    idx0 = edges_supervised[0].astype(jnp.int32)
    idx1 = edges_supervised[1].astype(jnp.int32)
    hf_n, hs_n = _normalize(h_first, h_second)
    return _sc_edge_dots(idx0, idx1, hf_n, hs_n)


# speedup vs baseline: 3.8431x; 3.8431x over previous
import functools

import jax
import jax.numpy as jnp
from jax import lax
from jax.experimental import pallas as pl
from jax.experimental.pallas import tpu as pltpu
from jax.experimental.pallas import tpu_sc as plsc

N_NODES = 10000
N_EDGES = 320000
D_FEAT = 128

NC = 2
NS = 16
L = 16
NW = NC * NS
EPW = N_EDGES // NW
CHUNK = 80
NGROUP = CHUNK // L
NCHUNK = EPW // CHUNK


def _normalize_body(hf_ref, hs_ref, of_ref, os_ref):
    hf = hf_ref[...]
    hs = hs_ref[...]
    of_ref[...] = hf * lax.rsqrt(jnp.sum(hf * hf, axis=1, keepdims=True))
    os_ref[...] = hs * lax.rsqrt(jnp.sum(hs * hs, axis=1, keepdims=True))


def _normalize(h_first, h_second):
    rows = h_first.shape[0]
    blk = 2000
    grid = rows // blk
    spec = pl.BlockSpec((blk, D_FEAT), lambda i: (i, 0))
    return pl.pallas_call(
        _normalize_body,
        grid=(grid,),
        in_specs=[spec, spec],
        out_specs=[spec, spec],
        out_shape=[
            jax.ShapeDtypeStruct(h_first.shape, jnp.float32),
            jax.ShapeDtypeStruct(h_second.shape, jnp.float32),
        ],
    )(h_first, h_second)


def _sc_body(idx0_hbm, idx1_hbm, hf_hbm, hs_hbm, out_hbm,
             i0_all, i1_all, out_all,
             ra0, rb0, ra1, rb1, sa0, sb0, sa1, sb1):
    wid = lax.axis_index("s") * NC + lax.axis_index("c")
    base = wid * EPW
    pltpu.sync_copy(idx0_hbm.at[pl.ds(base, EPW)], i0_all)
    pltpu.sync_copy(idx1_hbm.at[pl.ds(base, EPW)], i1_all)

    lanes = lax.iota(jnp.int32, L)

    def issue(c, ra, rb, sa, sb):
        ia = i0_all.at[pl.ds(c * CHUNK, CHUNK)]
        ib = i1_all.at[pl.ds(c * CHUNK, CHUNK)]
        pltpu.async_copy(hf_hbm.at[ia], ra, sa)
        pltpu.async_copy(hs_hbm.at[ib], rb, sb)

    def drain(c, ra, rb, sa, sb):
        ia = i0_all.at[pl.ds(c * CHUNK, CHUNK)]
        ib = i1_all.at[pl.ds(c * CHUNK, CHUNK)]
        pltpu.make_async_copy(hf_hbm.at[ia], ra, sa).wait()
        pltpu.make_async_copy(hs_hbm.at[ib], rb, sb).wait()

    def edge_dot(ra, rb, e):
        parts = []
        for j in range(0, D_FEAT, L):
            a = ra[e, pl.ds(j, L)]
            b = rb[e, pl.ds(j, L)]
            parts.append(a * b)
        while len(parts) > 1:
            parts = [parts[i] + parts[i + 1] for i in range(0, len(parts), 2)]
        return jnp.sum(parts[0])

    def compute(c, ra, rb):
        def group_body(g, carry2):
            e0 = g * L
            acc = jnp.zeros((L,), jnp.float32)
            for k in range(L):
                s = edge_dot(ra, rb, e0 + k)
                acc = jnp.where(lanes == k, jnp.full((L,), s, jnp.float32), acc)
            out_all[pl.ds(c * CHUNK + e0, L)] = acc
            return carry2

        lax.fori_loop(0, NGROUP, group_body, 0)

    issue(0, ra0, rb0, sa0, sb0)
    issue(1, ra1, rb1, sa1, sb1)

    def pair_body(cc, carry):
        c0 = 2 * cc
        drain(c0, ra0, rb0, sa0, sb0)
        compute(c0, ra0, rb0)

        @pl.when(c0 + 2 < NCHUNK)
        def _():
            issue(c0 + 2, ra0, rb0, sa0, sb0)

        drain(c0 + 1, ra1, rb1, sa1, sb1)
        compute(c0 + 1, ra1, rb1)

        @pl.when(c0 + 3 < NCHUNK)
        def _():
            issue(c0 + 3, ra1, rb1, sa1, sb1)

        return carry

    lax.fori_loop(0, NCHUNK // 2, pair_body, 0)
    if NCHUNK % 2:
        c_last = NCHUNK - 1
        drain(c_last, ra0, rb0, sa0, sb0)
        compute(c_last, ra0, rb0)

    pltpu.sync_copy(out_all, out_hbm.at[pl.ds(base, EPW)])


@functools.partial(jax.jit, static_argnames=())
def _sc_edge_dots(idx0, idx1, hf_n, hs_n):
    mesh = plsc.VectorSubcoreMesh(core_axis_name="c", subcore_axis_name="s")
    return pl.kernel(
        _sc_body,
        out_type=jax.ShapeDtypeStruct((N_EDGES,), jnp.float32),
        mesh=mesh,
        compiler_params=pltpu.CompilerParams(needs_layout_passes=False),
        scratch_types=[
            pltpu.VMEM((EPW,), jnp.int32),
            pltpu.VMEM((EPW,), jnp.int32),
            pltpu.VMEM((EPW,), jnp.float32),
            pltpu.VMEM((CHUNK, D_FEAT), jnp.float32),
            pltpu.VMEM((CHUNK, D_FEAT), jnp.float32),
            pltpu.VMEM((CHUNK, D_FEAT), jnp.float32),
            pltpu.VMEM((CHUNK, D_FEAT), jnp.float32),
            pltpu.SemaphoreType.DMA,
            pltpu.SemaphoreType.DMA,
            pltpu.SemaphoreType.DMA,
            pltpu.SemaphoreType.DMA,
        ],
    )(idx0, idx1, hf_n, hs_n)


def kernel(edges_supervised, h_first, h_second):
    idx0 = edges_supervised[0].astype(jnp.int32)
    idx1 = edges_supervised[1].astype(jnp.int32)
    hf_n, hs_n = _normalize(h_first, h_second)
    return _sc_edge_dots(idx0, idx1, hf_n, hs_n)


# speedup vs baseline: 5.3253x; 1.3857x over previous
import functools

import jax
import jax.numpy as jnp
from jax import lax
from jax.experimental import pallas as pl
from jax.experimental.pallas import tpu as pltpu
from jax.experimental.pallas import tpu_sc as plsc

N_NODES = 10000
N_EDGES = 320000
D_FEAT = 128

NC = 2
NS = 16
L = 16
NW = NC * NS
EPW = N_EDGES // NW
CHUNK = 80
NGROUP = CHUNK // L
NCHUNK = EPW // CHUNK


def _normalize_body(hf_ref, hs_ref, of_ref, os_ref):
    hf = hf_ref[...]
    hs = hs_ref[...]
    of_ref[...] = hf * lax.rsqrt(jnp.sum(hf * hf, axis=1, keepdims=True))
    os_ref[...] = hs * lax.rsqrt(jnp.sum(hs * hs, axis=1, keepdims=True))


def _normalize(h_first, h_second):
    rows = h_first.shape[0]
    blk = 2000
    grid = rows // blk
    spec = pl.BlockSpec((blk, D_FEAT), lambda i: (i, 0))
    return pl.pallas_call(
        _normalize_body,
        grid=(grid,),
        in_specs=[spec, spec],
        out_specs=[spec, spec],
        out_shape=[
            jax.ShapeDtypeStruct(h_first.shape, jnp.float32),
            jax.ShapeDtypeStruct(h_second.shape, jnp.float32),
        ],
    )(h_first, h_second)


def _sc_body(idx0_hbm, idx1_hbm, hf_hbm, hs_hbm, out_hbm,
             i0_all, i1_all, out_all, stage,
             ra0, rb0, ra1, rb1, sa0, sb0, sa1, sb1):
    wid = lax.axis_index("s") * NC + lax.axis_index("c")
    base = wid * EPW
    pltpu.sync_copy(idx0_hbm.at[pl.ds(base, EPW)], i0_all)
    pltpu.sync_copy(idx1_hbm.at[pl.ds(base, EPW)], i1_all)

    lanes = lax.iota(jnp.int32, L)

    def issue(c, ra, rb, sa, sb):
        ia = i0_all.at[pl.ds(c * CHUNK, CHUNK)]
        ib = i1_all.at[pl.ds(c * CHUNK, CHUNK)]
        pltpu.async_copy(hf_hbm.at[ia], ra, sa)
        pltpu.async_copy(hs_hbm.at[ib], rb, sb)

    def drain(c, ra, rb, sa, sb):
        ia = i0_all.at[pl.ds(c * CHUNK, CHUNK)]
        ib = i1_all.at[pl.ds(c * CHUNK, CHUNK)]
        pltpu.make_async_copy(hf_hbm.at[ia], ra, sa).wait()
        pltpu.make_async_copy(hs_hbm.at[ib], rb, sb).wait()

    def edge_partial(ra, rb, e):
        acc0 = ra[e, pl.ds(0, L)] * rb[e, pl.ds(0, L)]
        acc1 = ra[e, pl.ds(L, L)] * rb[e, pl.ds(L, L)]
        for j in range(2 * L, D_FEAT, 2 * L):
            acc0 = acc0 + ra[e, pl.ds(j, L)] * rb[e, pl.ds(j, L)]
            acc1 = acc1 + ra[e, pl.ds(j + L, L)] * rb[e, pl.ds(j + L, L)]
        return acc0 + acc1

    def compute(c, ra, rb):
        def group_body(g, carry2):
            e0 = g * L
            for k in range(L):
                s = edge_partial(ra, rb, e0 + k)
                col = jnp.full((L,), k, jnp.int32)
                plsc.store_scatter(stage, [lanes, col], s)
            parts = [stage[j, pl.ds(0, L)] for j in range(L)]
            while len(parts) > 1:
                parts = [parts[i] + parts[i + 1] for i in range(0, len(parts), 2)]
            out_all[pl.ds(c * CHUNK + e0, L)] = parts[0]
            return carry2

        lax.fori_loop(0, NGROUP, group_body, 0)

    issue(0, ra0, rb0, sa0, sb0)
    issue(1, ra1, rb1, sa1, sb1)

    def pair_body(cc, carry):
        c0 = 2 * cc
        drain(c0, ra0, rb0, sa0, sb0)
        compute(c0, ra0, rb0)

        @pl.when(c0 + 2 < NCHUNK)
        def _():
            issue(c0 + 2, ra0, rb0, sa0, sb0)

        drain(c0 + 1, ra1, rb1, sa1, sb1)
        compute(c0 + 1, ra1, rb1)

        @pl.when(c0 + 3 < NCHUNK)
        def _():
            issue(c0 + 3, ra1, rb1, sa1, sb1)

        return carry

    lax.fori_loop(0, NCHUNK // 2, pair_body, 0)
    if NCHUNK % 2:
        c_last = NCHUNK - 1
        drain(c_last, ra0, rb0, sa0, sb0)
        compute(c_last, ra0, rb0)

    pltpu.sync_copy(out_all, out_hbm.at[pl.ds(base, EPW)])


@functools.partial(jax.jit, static_argnames=())
def _sc_edge_dots(idx0, idx1, hf_n, hs_n):
    mesh = plsc.VectorSubcoreMesh(core_axis_name="c", subcore_axis_name="s")
    return pl.kernel(
        _sc_body,
        out_type=jax.ShapeDtypeStruct((N_EDGES,), jnp.float32),
        mesh=mesh,
        compiler_params=pltpu.CompilerParams(needs_layout_passes=False),
        scratch_types=[
            pltpu.VMEM((EPW,), jnp.int32),
            pltpu.VMEM((EPW,), jnp.int32),
            pltpu.VMEM((EPW,), jnp.float32),
            pltpu.VMEM((L, L + 1), jnp.float32),
            pltpu.VMEM((CHUNK, D_FEAT), jnp.float32),
            pltpu.VMEM((CHUNK, D_FEAT), jnp.float32),
            pltpu.VMEM((CHUNK, D_FEAT), jnp.float32),
            pltpu.VMEM((CHUNK, D_FEAT), jnp.float32),
            pltpu.SemaphoreType.DMA,
            pltpu.SemaphoreType.DMA,
            pltpu.SemaphoreType.DMA,
            pltpu.SemaphoreType.DMA,
        ],
    )(idx0, idx1, hf_n, hs_n)


def kernel(edges_supervised, h_first, h_second):
    idx0 = edges_supervised[0].astype(jnp.int32)
    idx1 = edges_supervised[1].astype(jnp.int32)
    hf_n, hs_n = _normalize(h_first, h_second)
    return _sc_edge_dots(idx0, idx1, hf_n, hs_n)
